# Initial kernel scaffold; baseline (speedup 1.0000x reference)
import jax
import jax.numpy as jnp
from jax.experimental import pallas as pl


def kernel(x_src, x_target, meta_edge_index, meta_edge_attr, W_att, b_att, W_tgt, b_tgt, att, bias):
    raise NotImplementedError("write your pallas kernel here")


# speedup vs baseline: 59.2429x; 59.2429x over previous
import functools

import jax
import jax.numpy as jnp
from jax import lax
from jax.experimental import pallas as pl
from jax.experimental.pallas import tpu as pltpu
from jax.experimental.pallas import tpu_sc as plsc

_N = 10000
_E = 320000
_DS = 128
_H = 8
_OD = 8
_HOD = _H * _OD

_NC = 2
_NS = 16
_NW = _NC * _NS
_CHUNK = _E // _NW
_B = 1000
_NB = _CHUNK // _B
_B2 = 500
_NB2 = _CHUNK // _B2
_RSTG = 1000
_NTS = _N // _RSTG
_RWB = 500

_BN = 2000
_BE2 = 4000
_BN2 = 1000


def _prep_body(xs_ref, xt_ref, w1_ref, w2_ref, wt_ref, bt_ref,
               as_ref, at_ref, v_ref):
    xs = xs_ref[...]
    xt = xt_ref[...]
    as_ref[...] = jnp.dot(xs, w1_ref[...], preferred_element_type=jnp.float32)
    at_ref[...] = jnp.dot(xt, w2_ref[...], preferred_element_type=jnp.float32)
    v_ref[...] = (jnp.dot(xt, wt_ref[...], preferred_element_type=jnp.float32)
                  + bt_ref[...])


def _prep(x_src, x_target, w1, w2, wt, bt):
    grid = (_N // _BN,)
    return pl.pallas_call(
        _prep_body,
        grid=grid,
        in_specs=[
            pl.BlockSpec((_BN, _DS), lambda i: (i, 0)),
            pl.BlockSpec((_BN, _DS), lambda i: (i, 0)),
            pl.BlockSpec((_DS, _HOD), lambda i: (0, 0)),
            pl.BlockSpec((_DS, _HOD), lambda i: (0, 0)),
            pl.BlockSpec((_DS, _OD), lambda i: (0, 0)),
            pl.BlockSpec((1, _OD), lambda i: (0, 0)),
        ],
        out_specs=[
            pl.BlockSpec((_BN, _HOD), lambda i: (i, 0)),
            pl.BlockSpec((_BN, _HOD), lambda i: (i, 0)),
            pl.BlockSpec((_BN, _OD), lambda i: (i, 0)),
        ],
        out_shape=[
            jax.ShapeDtypeStruct((_N, _HOD), jnp.float32),
            jax.ShapeDtypeStruct((_N, _HOD), jnp.float32),
            jax.ShapeDtypeStruct((_N, _OD), jnp.float32),
        ],
    )(x_src, x_target, w1, w2, wt, bt)


def _pass1_body(as_hbm, at_hbm, idxs_hbm, idxt_hbm,
                xsum_hbm,
                idxs_v, idxt_v, bufx, sem):
    c = lax.axis_index("c")
    s = lax.axis_index("s")
    wid = s * _NC + c
    base = wid * _CHUNK

    pltpu.sync_copy(idxs_hbm.at[wid], idxs_v)
    pltpu.sync_copy(idxt_hbm.at[wid], idxt_v)

    def step(j, carry):
        off = base + j * _B
        pltpu.async_copy(as_hbm.at[idxs_v.at[j]], bufx, sem).wait()
        pltpu.async_copy(at_hbm.at[idxt_v.at[j]], bufx, sem, add=True).wait()
        pltpu.sync_copy(bufx, xsum_hbm.at[pl.ds(off, _B)])
        return carry

    lax.fori_loop(0, _NB, step, 0)


def _pass1(a_src, a_tgt, idx_s3, idx_t3):
    mesh = plsc.VectorSubcoreMesh(core_axis_name="c", subcore_axis_name="s")
    f = pl.kernel(
        _pass1_body,
        out_type=jax.ShapeDtypeStruct((_E, _HOD), jnp.float32),
        mesh=mesh,
        compiler_params=pltpu.CompilerParams(use_tc_tiling_on_sc=False,
                                             needs_layout_passes=False),
        scratch_types=[
            pltpu.VMEM((_NB, _B), jnp.int32),
            pltpu.VMEM((_NB, _B), jnp.int32),
            pltpu.VMEM((_B, _HOD), jnp.float32),
            pltpu.SemaphoreType.DMA,
        ],
    )
    return f(a_src, a_tgt, idx_s3, idx_t3)


def _alpha_body(xsum_ref, ea_ref, w3d_ref, batt2_ref, attf2_ref, sel2_ref,
                alpha_ref, gmax_ref):
    i = pl.program_id(0)
    x = (xsum_ref[...]
         + jnp.dot(ea_ref[...], w3d_ref[...],
                   preferred_element_type=jnp.float32)
         + batt2_ref[...])
    xl = jnp.where(x >= 0.0, x, 0.2 * x) * attf2_ref[...]
    al = jnp.dot(xl, sel2_ref[...], preferred_element_type=jnp.float32)
    alpha_ref[...] = al
    m = jnp.max(al)

    @pl.when(i == 0)
    def _():
        gmax_ref[...] = jnp.full((1, 16), m, jnp.float32)

    @pl.when(i > 0)
    def _():
        gmax_ref[...] = jnp.maximum(gmax_ref[...], m)


def _alpha(xsum128, ea8, w3d, batt2, attf2, sel2):
    grid = (_E // 2 // _BE2,)
    return pl.pallas_call(
        _alpha_body,
        grid=grid,
        in_specs=[
            pl.BlockSpec((_BE2, 2 * _HOD), lambda i: (i, 0)),
            pl.BlockSpec((_BE2, 8), lambda i: (i, 0)),
            pl.BlockSpec((8, 2 * _HOD), lambda i: (0, 0)),
            pl.BlockSpec((1, 2 * _HOD), lambda i: (0, 0)),
            pl.BlockSpec((1, 2 * _HOD), lambda i: (0, 0)),
            pl.BlockSpec((2 * _HOD, 16), lambda i: (0, 0)),
        ],
        out_specs=[
            pl.BlockSpec((_BE2, 16), lambda i: (i, 0)),
            pl.BlockSpec((1, 16), lambda i: (0, 0)),
        ],
        out_shape=[
            jax.ShapeDtypeStruct((_E // 2, 16), jnp.float32),
            jax.ShapeDtypeStruct((1, 16), jnp.float32),
        ],
    )(xsum128, ea8, w3d, batt2, attf2, sel2)


def _pgather(x, idx):
    return lax.gather(
        x, idx[:, None],
        lax.GatherDimensionNumbers(
            offset_dims=(), collapsed_slice_dims=(0,), start_index_map=(0,)),
        (1,),
        mode=lax.GatherScatterMode.PROMISE_IN_BOUNDS)


def _pass2_body(alpha_hbm, gmax_hbm, v_hbm, idxs_hbm, idxt_hbm,
                outp_hbm, outd_hbm,
                idxs_b, idxt_b, albuf, vgbuf, mp, md, gbuf,
                accp, accd, vtab, sem):
    c = lax.axis_index("c")
    s = lax.axis_index("s")
    wid = s * _NC + c

    zeros16 = jnp.zeros((16,), jnp.float32)
    lanes = lax.iota(jnp.int32, 16)
    rowpat = lax.shift_right_logical(lanes, 3)
    colpat = lanes & 7
    eb_idx = [rowpat + 2 * k for k in range(4)]

    def zp(i, carry):
        for k in range(4):
            mp[i, pl.ds(16 * k, 16)] = zeros16
        return carry
    lax.fori_loop(0, _B2, zp, 0)

    def zd(i, carry):
        plsc.store_scatter(md.at[pl.ds(2 * i, 2)], [rowpat, colpat], zeros16)
        return carry
    lax.fori_loop(0, _B2 // 2, zd, 0)

    r0 = s * _RSTG

    @pl.when(s < _NTS)
    def _():
        pltpu.sync_copy(v_hbm.at[pl.ds(r0, _RSTG)],
                        vtab.at[pl.ds(r0, _RSTG)])
        for k in range(2):
            pltpu.sync_copy(mp, accp.at[pl.ds(r0 + k * _RWB, _RWB)])
            pltpu.sync_copy(md, accd.at[pl.ds(r0 + k * _RWB, _RWB)])

    pltpu.sync_copy(gmax_hbm, gbuf)
    gv = gbuf[...]
    plsc.subcore_barrier()

    def block(j, carry):
        off8 = (wid * _CHUNK + j * _B2) * 8
        pltpu.sync_copy(alpha_hbm.at[pl.ds(off8, _B2 * 8)], albuf)
        pltpu.sync_copy(idxs_hbm.at[wid, pl.ds(j, 1)], idxs_b)
        pltpu.sync_copy(idxt_hbm.at[wid, pl.ds(j, 1)], idxt_b)
        pltpu.async_copy(vtab.at[idxt_b.at[0]], vgbuf, sem).wait()

        def pair(i, carry2):
            av = albuf[pl.ds(16 * i, 16)]
            ex = jnp.exp(av - gv)
            ra = 2 * i
            rb = 2 * i + 1
            vv = plsc.load_gather(vgbuf.at[pl.ds(ra, 2)], [rowpat, colpat])
            vta = _pgather(vv, colpat)
            vtb = _pgather(vv, colpat + 8)
            for k in range(4):
                mp[ra, pl.ds(16 * k, 16)] = _pgather(ex, eb_idx[k]) * vta
                mp[rb, pl.ds(16 * k, 16)] = _pgather(ex, eb_idx[k] + 8) * vtb
            plsc.store_scatter(md.at[pl.ds(ra, 2)], [rowpat, colpat], ex)
            return carry2

        lax.fori_loop(0, _B2 // 2, pair, 0)
        pltpu.sync_copy(mp, accp.at[idxs_b.at[0]], add=True)
        pltpu.sync_copy(md, accd.at[idxs_b.at[0]], add=True)
        return carry

    lax.fori_loop(0, _NB2, block, 0)

    plsc.subcore_barrier()

    @pl.when(s < _NTS)
    def _():
        zeroi = jnp.zeros((16,), jnp.int32)
        for k in range(2):
            rk = r0 + k * _RWB
            pltpu.sync_copy(accp.at[pl.ds(rk, _RWB)],
                            outp_hbm.at[c, pl.ds(rk, _RWB)])
            pltpu.sync_copy(accd.at[pl.ds(rk, _RWB)], vgbuf)

            def expand(r, carry):
                for q in range(4):
                    mp[r, pl.ds(16 * q, 16)] = plsc.load_gather(
                        vgbuf.at[pl.ds(r, 1)], [zeroi, rowpat + 2 * q])
                return carry

            lax.fori_loop(0, _RWB, expand, 0)
            pltpu.sync_copy(mp, outd_hbm.at[c, pl.ds(rk, _RWB)])


def _pass2(alpha_flat, gmax16, v, idx_s3, idx_t3):
    mesh = plsc.VectorSubcoreMesh(core_axis_name="c", subcore_axis_name="s")
    f = pl.kernel(
        _pass2_body,
        compiler_params=pltpu.CompilerParams(use_tc_tiling_on_sc=False,
                                             needs_layout_passes=False),
        out_type=[
            jax.ShapeDtypeStruct((_NC, _N, _HOD), jnp.float32),
            jax.ShapeDtypeStruct((_NC, _N, _HOD), jnp.float32),
        ],
        mesh=mesh,
        scratch_types=[
            pltpu.VMEM((1, _B2), jnp.int32),
            pltpu.VMEM((1, _B2), jnp.int32),
            pltpu.VMEM((_B2 * 8,), jnp.float32),
            pltpu.VMEM((_B2, _OD), jnp.float32),
            pltpu.VMEM((_B2, _HOD), jnp.float32),
            pltpu.VMEM((_B2, _OD), jnp.float32),
            pltpu.VMEM((16,), jnp.float32),
            pltpu.VMEM_SHARED((_N, _HOD), jnp.float32),
            pltpu.VMEM_SHARED((_N, _OD), jnp.float32),
            pltpu.VMEM_SHARED((_N, _OD), jnp.float32),
            pltpu.SemaphoreType.DMA,
        ],
    )
    return f(alpha_flat, gmax16, v, idx_s3, idx_t3)


def _final_body(p_ref, d_ref, bias2_ref, out_ref):
    p = p_ref[0] + p_ref[1]
    db = d_ref[0] + d_ref[1]
    out_ref[...] = jnp.where(db > 0.0, p / db, 0.0) + bias2_ref[...]


def _finalize(outp2, outd2, bias2):
    grid = (_N // 2 // _BN2,)
    return pl.pallas_call(
        _final_body,
        grid=grid,
        in_specs=[
            pl.BlockSpec((_NC, _BN2, 2 * _HOD), lambda i: (0, i, 0)),
            pl.BlockSpec((_NC, _BN2, 2 * _HOD), lambda i: (0, i, 0)),
            pl.BlockSpec((1, 2 * _HOD), lambda i: (0, 0)),
        ],
        out_specs=pl.BlockSpec((_BN2, 2 * _HOD), lambda i: (i, 0)),
        out_shape=jax.ShapeDtypeStruct((_N // 2, 2 * _HOD), jnp.float32),
    )(outp2, outd2, bias2)


@jax.jit
def kernel(x_src, x_target, meta_edge_index, meta_edge_attr,
           W_att, b_att, W_tgt, b_tgt, att, bias):
    w1 = W_att[:_DS]
    w2 = W_att[_DS:2 * _DS]
    w3 = W_att[2 * _DS:]
    a_src, a_tgt, v = _prep(x_src, x_target, w1, w2, W_tgt,
                            b_tgt.reshape(1, _OD))

    idx_s3 = meta_edge_index[0].reshape(_NW, _NB, _B)
    idx_t3 = meta_edge_index[1].reshape(_NW, _NB, _B)
    xsum = _pass1(a_src, a_tgt, idx_s3, idx_t3)

    xsum128 = xsum.reshape(_E // 2, 2 * _HOD)
    ea8 = meta_edge_attr.reshape(_E // 2, 8)
    zero4 = jnp.zeros((4, _HOD), jnp.float32)
    w3d = jnp.concatenate(
        [jnp.concatenate([w3, zero4], axis=1),
         jnp.concatenate([zero4, w3], axis=1)], axis=0)
    batt2 = jnp.tile(b_att.reshape(1, _HOD), (1, 2))
    attf2 = jnp.tile(att.reshape(1, _HOD), (1, 2))
    sel = (jnp.arange(_HOD)[:, None] // _OD
           == jnp.arange(_H)[None, :]).astype(jnp.float32)
    zero64 = jnp.zeros((_HOD, _H), jnp.float32)
    sel2 = jnp.concatenate(
        [jnp.concatenate([sel, zero64], axis=1),
         jnp.concatenate([zero64, sel], axis=1)], axis=0)

    alpha, gmax = _alpha(xsum128, ea8, w3d, batt2, attf2, sel2)

    idx_s3b = meta_edge_index[0].reshape(_NW, _NB2, _B2)
    idx_t3b = meta_edge_index[1].reshape(_NW, _NB2, _B2)
    outp, outd = _pass2(alpha.reshape(_E * 8), gmax.reshape(16), v,
                        idx_s3b, idx_t3b)

    out2 = _finalize(outp.reshape(_NC, _N // 2, 2 * _HOD),
                     outd.reshape(_NC, _N // 2, 2 * _HOD),
                     jnp.tile(bias.reshape(1, _HOD), (1, 2)))
    return out2.reshape(_N, _HOD)
